# baseline (speedup 1.0000x reference)
import jax
import jax.numpy as jnp
from jax import lax
from jax.experimental import pallas as pl
from jax.experimental.pallas import tpu as pltpu, tpu_sc as plsc

B = 16384
D = 64
NC = 2
NS = 16
NW = NC * NS
ROWS_PER_W = B // NW
ELEMS_PER_W = ROWS_PER_W * D
GROUPS = ROWS_PER_W // 16


def _body(gu_h, gi_h, out_h, gu_v, gi_v, out_v):
    wid = lax.axis_index("s") * NC + lax.axis_index("c")
    ebase = wid * ELEMS_PER_W
    pltpu.sync_copy(gu_h.at[pl.ds(ebase, ELEMS_PER_W)], gu_v)
    pltpu.sync_copy(gi_h.at[pl.ds(ebase, ELEMS_PER_W)], gi_v)

    lane_off = lax.iota(jnp.int32, 16) * D

    def group(g, carry):
        base_idx = lane_off + g * (16 * D)
        acc = jnp.zeros((16,), jnp.float32)
        for d in range(D):
            idx = base_idx + d
            a = plsc.load_gather(gu_v, [idx])
            b = plsc.load_gather(gi_v, [idx])
            acc = acc + a * b
        out_v[pl.ds(g * 16, 16)] = acc
        return carry

    lax.fori_loop(0, GROUPS, group, 0)
    pltpu.sync_copy(out_v, out_h.at[pl.ds(wid * ROWS_PER_W, ROWS_PER_W)])


_sc_call = pl.kernel(
    _body,
    out_type=jax.ShapeDtypeStruct((B,), jnp.float32),
    mesh=plsc.VectorSubcoreMesh(core_axis_name="c", subcore_axis_name="s"),
    compiler_params=pltpu.CompilerParams(needs_layout_passes=False),
    scratch_types=[
        pltpu.VMEM((ELEMS_PER_W,), jnp.float32),
        pltpu.VMEM((ELEMS_PER_W,), jnp.float32),
        pltpu.VMEM((ROWS_PER_W,), jnp.float32),
    ],
)


@jax.jit
def kernel(gu, gi):
    return _sc_call(gu.reshape(-1), gi.reshape(-1))


# speedup vs baseline: 2.9249x; 2.9249x over previous
import jax
import jax.numpy as jnp
from jax.experimental import pallas as pl

B = 16384
D = 64
BLK = 1024
NB = B // BLK


def _body(gu_ref, gi_ref, o_ref):
    o_ref[...] = jnp.sum(gu_ref[...] * gi_ref[...], axis=1)


_call = pl.pallas_call(
    _body,
    grid=(NB,),
    in_specs=[
        pl.BlockSpec((BLK, D), lambda i: (i, 0)),
        pl.BlockSpec((BLK, D), lambda i: (i, 0)),
    ],
    out_specs=pl.BlockSpec((BLK,), lambda i: (i,)),
    out_shape=jax.ShapeDtypeStruct((B,), jnp.float32),
)


@jax.jit
def kernel(gu, gi):
    return _call(gu, gi)


# speedup vs baseline: 3.0917x; 1.0570x over previous
import jax
import jax.numpy as jnp
from jax.experimental import pallas as pl
from jax.experimental.pallas import tpu as pltpu

B = 16384
D = 64
BLK = 1024
NB = B // BLK


def _body(gu_ref, gi_ref, o_ref):
    i = pl.program_id(0)
    p = gu_ref[...] * gi_ref[...]
    pt = p.T
    ones = jnp.ones((8, D), jnp.float32)
    rv = jax.lax.dot_general(
        ones, pt, (((1,), (0,)), ((), ())),
        preferred_element_type=jnp.float32,
    )
    o_ref[i, :] = rv[0, :]


_call = pl.pallas_call(
    _body,
    grid=(NB,),
    in_specs=[
        pl.BlockSpec((BLK, D), lambda i: (i, 0)),
        pl.BlockSpec((BLK, D), lambda i: (i, 0)),
    ],
    out_specs=pl.BlockSpec((NB, BLK), lambda i: (0, 0)),
    out_shape=jax.ShapeDtypeStruct((NB, BLK), jnp.float32),
    compiler_params=pltpu.CompilerParams(
        dimension_semantics=("arbitrary",),
    ),
)


@jax.jit
def kernel(gu, gi):
    return _call(gu, gi).reshape(B)


# speedup vs baseline: 12.0063x; 3.8834x over previous
import jax
import jax.numpy as jnp
from jax.experimental import pallas as pl
from jax.experimental.pallas import tpu as pltpu

B = 16384
D = 64
BLK = 2048
NB = B // BLK


def _body(gu_ref, gi_ref, o_ref):
    o_ref[...] = jnp.sum(gu_ref[...] * gi_ref[...], axis=0)


_call = pl.pallas_call(
    _body,
    grid=(NB,),
    in_specs=[
        pl.BlockSpec((D, BLK), lambda i: (0, i)),
        pl.BlockSpec((D, BLK), lambda i: (0, i)),
    ],
    out_specs=pl.BlockSpec((BLK,), lambda i: (i,)),
    out_shape=jax.ShapeDtypeStruct((B,), jnp.float32),
    compiler_params=pltpu.CompilerParams(
        dimension_semantics=("arbitrary",),
    ),
)


@jax.jit
def kernel(gu, gi):
    return _call(gu.T, gi.T)


# speedup vs baseline: 12.0187x; 1.0010x over previous
import jax
import jax.numpy as jnp
from jax.experimental import pallas as pl
from jax.experimental.pallas import tpu as pltpu

B = 16384
D = 64
BLK = 2048
NB = B // BLK


def _body(gu_ref, gi_ref, o_ref):
    o_ref[...] = jnp.sum(gu_ref[...] * gi_ref[...], axis=0)


_call = pl.pallas_call(
    _body,
    grid=(NB,),
    in_specs=[
        pl.BlockSpec((D, BLK), lambda i: (0, i)),
        pl.BlockSpec((D, BLK), lambda i: (0, i)),
    ],
    out_specs=pl.BlockSpec((BLK,), lambda i: (i,)),
    out_shape=jax.ShapeDtypeStruct((B,), jnp.float32),
    compiler_params=pltpu.CompilerParams(
        dimension_semantics=("arbitrary",),
    ),
)


@jax.jit
def kernel(gu, gi):
    gut = pltpu.with_memory_space_constraint(gu.T, pltpu.MemorySpace.HBM)
    git = pltpu.with_memory_space_constraint(gi.T, pltpu.MemorySpace.HBM)
    return _call(gut, git)


# speedup vs baseline: 16.6382x; 1.3844x over previous
import jax
import jax.numpy as jnp
from jax.experimental import pallas as pl
from jax.experimental.pallas import tpu as pltpu

B = 16384
D = 64
BLK = 4096
NB = B // BLK


def _body(gu_ref, gi_ref, o_ref):
    o_ref[...] = jnp.sum(gu_ref[...] * gi_ref[...], axis=0)


_call = pl.pallas_call(
    _body,
    grid=(NB,),
    in_specs=[
        pl.BlockSpec((D, BLK), lambda i: (0, i)),
        pl.BlockSpec((D, BLK), lambda i: (0, i)),
    ],
    out_specs=pl.BlockSpec((BLK,), lambda i: (i,)),
    out_shape=jax.ShapeDtypeStruct((B,), jnp.float32),
    compiler_params=pltpu.CompilerParams(
        dimension_semantics=("arbitrary",),
    ),
)


@jax.jit
def kernel(gu, gi):
    gut = pltpu.with_memory_space_constraint(gu.T, pltpu.MemorySpace.HBM)
    git = pltpu.with_memory_space_constraint(gi.T, pltpu.MemorySpace.HBM)
    return _call(gut, git)


# speedup vs baseline: 18.3068x; 1.1003x over previous
import jax
import jax.numpy as jnp
from jax.experimental import pallas as pl
from jax.experimental.pallas import tpu as pltpu

B = 16384
D = 64
BLK = 8192
NB = B // BLK


def _body(gu_ref, gi_ref, o_ref):
    o_ref[...] = jnp.sum(gu_ref[...] * gi_ref[...], axis=0)


_call = pl.pallas_call(
    _body,
    grid=(NB,),
    in_specs=[
        pl.BlockSpec((D, BLK), lambda i: (0, i)),
        pl.BlockSpec((D, BLK), lambda i: (0, i)),
    ],
    out_specs=pl.BlockSpec((BLK,), lambda i: (i,)),
    out_shape=jax.ShapeDtypeStruct((B,), jnp.float32),
    compiler_params=pltpu.CompilerParams(
        dimension_semantics=("arbitrary",),
    ),
)


@jax.jit
def kernel(gu, gi):
    gut = pltpu.with_memory_space_constraint(gu.T, pltpu.MemorySpace.HBM)
    git = pltpu.with_memory_space_constraint(gi.T, pltpu.MemorySpace.HBM)
    return _call(gut, git)
